# baseline (speedup 1.0000x reference)
import jax
import jax.numpy as jnp
from jax import lax
from jax.experimental import pallas as pl
from jax.experimental.pallas import tpu as pltpu
from jax.experimental.pallas import tpu_sc as plsc

_NUM_CORES = 2
_NUM_SUBCORES = 16
_NUM_WORKERS = _NUM_CORES * _NUM_SUBCORES
_LANES = 16
_CHUNK = 128
_SLOTS = 4


def _sc_aggregate(x, sd4, w3, n_rows):
    feat = x.shape[1]
    n_chunks = sd4.shape[1]
    rows_per_tile = n_rows // _NUM_SUBCORES
    jblocks = feat // _LANES

    mesh = plsc.VectorSubcoreMesh(
        core_axis_name="c", subcore_axis_name="s",
        num_cores=_NUM_CORES, num_subcores=_NUM_SUBCORES)

    def body(x_hbm, sd_hbm, w_hbm, out_hbm,
             acc, ws_v, sd0, sd1, sd2, sd3, rows0, rows1,
             wsem, gsem0, gsem1, ssem0, ssem1, ssem2, ssem3):
        cid = lax.axis_index("c")
        sid = lax.axis_index("s")
        wid = sid * _NUM_CORES + cid
        sd_slots = (sd0, sd1, sd2, sd3)
        sd_sems = (ssem0, ssem1, ssem2, ssem3)
        rows = (rows0, rows1)
        gsems = (gsem0, gsem1)

        ld_w = pltpu.async_copy(w_hbm.at[wid], ws_v, wsem)

        def load_sd(ci, t):
            pltpu.async_copy(sd_hbm.at[wid, ci], sd_slots[t], sd_sems[t])

        def wait_sd(ci, t):
            pltpu.make_async_copy(
                sd_hbm.at[wid, ci], sd_slots[t], sd_sems[t]).wait()

        def start_gather(ci, t, b):
            pltpu.async_copy(x_hbm.at[sd_slots[t].at[0]], rows[b], gsems[b])

        def wait_gather(ci, t, b):
            pltpu.make_async_copy(
                x_hbm.at[sd_slots[t].at[0]], rows[b], gsems[b]).wait()

        for t in range(_SLOTS):
            load_sd(t, t)

        zvec = jnp.zeros((_LANES,), jnp.float32)

        def zfill(i, _):
            for j in range(jblocks):
                rows0[i, pl.ds(j * _LANES, _LANES)] = zvec
            return 0

        lax.fori_loop(0, _CHUNK, zfill, 0)
        row0 = sid * rows_per_tile
        for k in range(rows_per_tile // _CHUNK):
            pltpu.sync_copy(rows0, acc.at[pl.ds(row0 + k * _CHUNK, _CHUNK)])
        ld_w.wait()
        plsc.subcore_barrier()
        wait_sd(0, 0)
        start_gather(0, 0, 0)

        def scale(ci, b):
            buf = rows[b]

            def sbody(g, _):
                w16 = ws_v[ci, pl.ds(g * _LANES, _LANES)]
                for k in range(_LANES):
                    we = w16[k]
                    e = g * _LANES + k
                    for j in range(jblocks):
                        sl = pl.ds(j * _LANES, _LANES)
                        buf[e, sl] = buf[e, sl] * we
                return 0

            lax.fori_loop(0, _CHUNK // _LANES, sbody, 0)

        def quad(q, _):
            c0 = q * _SLOTS
            for t in range(_SLOTS):
                ct = c0 + t
                b = t % 2
                if t < _SLOTS - 1:
                    wait_sd(ct + 1, t + 1)
                    start_gather(ct + 1, t + 1, 1 - b)
                else:
                    @pl.when(ct + 1 < n_chunks)
                    def _():
                        wait_sd(ct + 1, 0)
                        start_gather(ct + 1, 0, 1 - b)

                wait_gather(ct, t, b)
                scale(ct, b)
                pltpu.sync_copy(rows[b], acc.at[sd_slots[t].at[1]], add=True)

                @pl.when(ct + _SLOTS < n_chunks)
                def _():
                    load_sd(ct + _SLOTS, t)
            return 0

        lax.fori_loop(0, n_chunks // _SLOTS, quad, 0)
        plsc.subcore_barrier()

        pltpu.sync_copy(acc.at[pl.ds(row0, rows_per_tile)],
                        out_hbm.at[cid, pl.ds(row0, rows_per_tile)])

    fn = pl.kernel(
        body,
        out_type=jax.ShapeDtypeStruct((_NUM_CORES, n_rows, feat),
                                      jnp.float32),
        mesh=mesh,
        scratch_types=[
            pltpu.VMEM_SHARED((n_rows, feat), jnp.float32),
            pltpu.VMEM((n_chunks, _CHUNK), jnp.float32),
            pltpu.VMEM((2, _CHUNK), jnp.int32),
            pltpu.VMEM((2, _CHUNK), jnp.int32),
            pltpu.VMEM((2, _CHUNK), jnp.int32),
            pltpu.VMEM((2, _CHUNK), jnp.int32),
            pltpu.VMEM((_CHUNK, feat), jnp.float32),
            pltpu.VMEM((_CHUNK, feat), jnp.float32),
            pltpu.SemaphoreType.DMA,
            pltpu.SemaphoreType.DMA,
            pltpu.SemaphoreType.DMA,
            pltpu.SemaphoreType.DMA,
            pltpu.SemaphoreType.DMA,
            pltpu.SemaphoreType.DMA,
            pltpu.SemaphoreType.DMA,
        ],
    )
    return fn(x, sd4, w3)


def _tc_matmul_relu(acc, W):
    n_rows, feat = acc.shape[1], acc.shape[2]
    out_f = W.shape[1]
    block = 1024
    grid = n_rows // block

    def body(a_ref, w_ref, o_ref):
        s = a_ref[0] + a_ref[1]
        o_ref[...] = jnp.maximum(
            jnp.dot(s, w_ref[...], preferred_element_type=jnp.float32), 0.0)

    return pl.pallas_call(
        body,
        grid=(grid,),
        in_specs=[
            pl.BlockSpec((_NUM_CORES, block, feat), lambda i: (0, i, 0)),
            pl.BlockSpec((feat, out_f), lambda i: (0, 0)),
        ],
        out_specs=pl.BlockSpec((block, out_f), lambda i: (i, 0)),
        out_shape=jax.ShapeDtypeStruct((n_rows, out_f), jnp.float32),
    )(acc, W)


def kernel(x, edge_index, edge_weight, W):
    n_nodes = x.shape[0]
    n_edges = edge_index.shape[1]
    grain = _NUM_WORKERS * _CHUNK * _SLOTS
    e_pad = -(-n_edges // grain) * grain
    pad = e_pad - n_edges
    src = jnp.concatenate([edge_index[0], jnp.zeros((pad,), jnp.int32)])
    dst = jnp.concatenate([edge_index[1], jnp.zeros((pad,), jnp.int32)])
    w = jnp.concatenate([edge_weight, jnp.zeros((pad,), jnp.float32)])
    n_chunks = e_pad // (_NUM_WORKERS * _CHUNK)
    shape3 = (_NUM_WORKERS, n_chunks, _CHUNK)
    sd4 = jnp.stack([src.reshape(shape3), dst.reshape(shape3)], axis=2)
    rgrain = _NUM_SUBCORES * _CHUNK
    n_rows = -(-n_nodes // rgrain) * rgrain
    acc = _sc_aggregate(x, sd4, w.reshape(shape3), n_rows)
    out = _tc_matmul_relu(acc, W)
    return out[:n_nodes]


# speedup vs baseline: 1.0908x; 1.0908x over previous
import jax
import jax.numpy as jnp
from jax import lax
from jax.experimental import pallas as pl
from jax.experimental.pallas import tpu as pltpu
from jax.experimental.pallas import tpu_sc as plsc

_NUM_CORES = 2
_NUM_SUBCORES = 16
_LANES = 16
_CHUNK = 128
_SLOTS = 4


def _sc_aggregate(x2p, sdw, n_pairs, feat):
    n_chunks = sdw.shape[1]
    pairs_per_tile = n_pairs // _NUM_SUBCORES
    half = feat // 2
    jblocks = half // _LANES

    mesh = plsc.VectorSubcoreMesh(
        core_axis_name="c", subcore_axis_name="s",
        num_cores=_NUM_CORES, num_subcores=_NUM_SUBCORES)

    def body(x_hbm, sdw_hbm, out_hbm,
             xs, acc, sl0, sl1, sl2, sl3, rows0, rows1,
             gsem0, gsem1, ssem0, ssem1, ssem2, ssem3):
        cid = lax.axis_index("c")
        sid = lax.axis_index("s")
        slots = (sl0, sl1, sl2, sl3)
        slot_sems = (ssem0, ssem1, ssem2, ssem3)
        rows = (rows0, rows1)
        gsems = (gsem0, gsem1)

        def load_sdw(ci, t):
            pltpu.async_copy(sdw_hbm.at[sid, ci], slots[t], slot_sems[t])

        def wait_sdw(ci, t):
            pltpu.make_async_copy(
                sdw_hbm.at[sid, ci], slots[t], slot_sems[t]).wait()

        def start_gather(t, b):
            pltpu.async_copy(xs.at[slots[t].at[0]], rows[b], gsems[b])

        def wait_gather(t, b):
            pltpu.make_async_copy(
                xs.at[slots[t].at[0]], rows[b], gsems[b]).wait()

        for t in range(_SLOTS):
            load_sdw(t, t)

        row0 = sid * pairs_per_tile
        pltpu.sync_copy(x_hbm.at[cid, pl.ds(row0, pairs_per_tile)],
                        xs.at[pl.ds(row0, pairs_per_tile)])

        zvec = jnp.zeros((_LANES,), jnp.float32)

        def zfill(i, _):
            for j in range(feat // _LANES):
                rows0[i, pl.ds(j * _LANES, _LANES)] = zvec
            return 0

        lax.fori_loop(0, _CHUNK, zfill, 0)
        full, rem = divmod(pairs_per_tile, _CHUNK)
        for k in range(full):
            pltpu.sync_copy(rows0, acc.at[pl.ds(row0 + k * _CHUNK, _CHUNK)])
        if rem:
            pltpu.sync_copy(rows0.at[pl.ds(0, rem)],
                            acc.at[pl.ds(row0 + full * _CHUNK, rem)])
        plsc.subcore_barrier()
        wait_sdw(0, 0)
        start_gather(0, 0)

        def scale(t, b):
            buf = rows[b]
            wrow = slots[t]

            def sbody(g, _):
                w16 = lax.bitcast_convert_type(
                    wrow[2, pl.ds(g * _LANES, _LANES)], jnp.float32)
                sp16 = wrow[3, pl.ds(g * _LANES, _LANES)]
                dp16 = wrow[4, pl.ds(g * _LANES, _LANES)]
                for k in range(_LANES):
                    e = g * _LANES + k
                    we = w16[k]
                    sodd = sp16[k] != 0
                    dodd = dp16[k] != 0
                    msg = []
                    for j in range(jblocks):
                        a = buf[e, pl.ds(j * _LANES, _LANES)]
                        bb = buf[e, pl.ds(half + j * _LANES, _LANES)]
                        msg.append(jnp.where(sodd, bb, a) * we)
                    for j in range(jblocks):
                        buf[e, pl.ds(j * _LANES, _LANES)] = (
                            jnp.where(dodd, zvec, msg[j]))
                        buf[e, pl.ds(half + j * _LANES, _LANES)] = (
                            jnp.where(dodd, msg[j], zvec))
                return 0

            lax.fori_loop(0, _CHUNK // _LANES, sbody, 0)

        def quad(q, _):
            c0 = q * _SLOTS
            for t in range(_SLOTS):
                ct = c0 + t
                b = t % 2
                if t < _SLOTS - 1:
                    wait_sdw(ct + 1, t + 1)
                    start_gather(t + 1, 1 - b)
                else:
                    @pl.when(ct + 1 < n_chunks)
                    def _():
                        wait_sdw(ct + 1, 0)
                        start_gather(0, 1 - b)

                wait_gather(t, b)
                scale(t, b)
                pltpu.sync_copy(rows[b], acc.at[slots[t].at[1]], add=True)

                @pl.when(ct + _SLOTS < n_chunks)
                def _():
                    load_sdw(ct + _SLOTS, t)
            return 0

        lax.fori_loop(0, n_chunks // _SLOTS, quad, 0)
        plsc.subcore_barrier()

        pltpu.sync_copy(acc.at[pl.ds(row0, pairs_per_tile)],
                        out_hbm.at[cid, pl.ds(row0, pairs_per_tile)])

    fn = pl.kernel(
        body,
        out_type=jax.ShapeDtypeStruct((_NUM_CORES, n_pairs, feat),
                                      jnp.float32),
        mesh=mesh,
        scratch_types=[
            pltpu.VMEM_SHARED((n_pairs, feat), jnp.float32),
            pltpu.VMEM_SHARED((n_pairs, feat), jnp.float32),
            pltpu.VMEM((5, _CHUNK), jnp.int32),
            pltpu.VMEM((5, _CHUNK), jnp.int32),
            pltpu.VMEM((5, _CHUNK), jnp.int32),
            pltpu.VMEM((5, _CHUNK), jnp.int32),
            pltpu.VMEM((_CHUNK, feat), jnp.float32),
            pltpu.VMEM((_CHUNK, feat), jnp.float32),
            pltpu.SemaphoreType.DMA,
            pltpu.SemaphoreType.DMA,
            pltpu.SemaphoreType.DMA,
            pltpu.SemaphoreType.DMA,
            pltpu.SemaphoreType.DMA,
            pltpu.SemaphoreType.DMA,
        ],
    )
    return fn(x2p, sdw)


def _tc_matmul_relu(acc, W2):
    n_rows, feat_half = acc.shape[1], acc.shape[2]
    out_f = W2.shape[2]
    block = 1024
    grid = n_rows // block

    def body(a_ref, w_ref, o_ref):
        o_ref[...] = jnp.maximum(
            jnp.dot(a_ref[0], w_ref[0], preferred_element_type=jnp.float32)
            + jnp.dot(a_ref[1], w_ref[1], preferred_element_type=jnp.float32),
            0.0)

    return pl.pallas_call(
        body,
        grid=(grid,),
        in_specs=[
            pl.BlockSpec((_NUM_CORES, block, feat_half), lambda i: (0, i, 0)),
            pl.BlockSpec((_NUM_CORES, feat_half, out_f), lambda i: (0, 0, 0)),
        ],
        out_specs=pl.BlockSpec((block, out_f), lambda i: (i, 0)),
        out_shape=jax.ShapeDtypeStruct((n_rows, out_f), jnp.float32),
    )(acc, W2)


def kernel(x, edge_index, edge_weight, W):
    n_nodes, feat = x.shape
    n_edges = edge_index.shape[1]
    feat_half = feat // _NUM_CORES
    grain = _NUM_SUBCORES * _CHUNK * _SLOTS
    e_pad = -(-n_edges // grain) * grain
    pad = e_pad - n_edges
    src = jnp.concatenate([edge_index[0], jnp.zeros((pad,), jnp.int32)])
    dst = jnp.concatenate([edge_index[1], jnp.zeros((pad,), jnp.int32)])
    w = jnp.concatenate([edge_weight, jnp.zeros((pad,), jnp.float32)])
    n_chunks = e_pad // (_NUM_SUBCORES * _CHUNK)
    shape3 = (_NUM_SUBCORES, n_chunks, _CHUNK)
    sdw = jnp.stack([
        (src >> 1).reshape(shape3),
        (dst >> 1).reshape(shape3),
        lax.bitcast_convert_type(w, jnp.int32).reshape(shape3),
        (src & 1).reshape(shape3),
        (dst & 1).reshape(shape3),
    ], axis=2)
    rgrain = _NUM_SUBCORES * _CHUNK
    n_rows = -(-n_nodes // rgrain) * rgrain
    n_pairs = n_rows // 2
    x2p = jnp.pad(
        x.reshape(n_nodes, _NUM_CORES, feat_half).transpose(1, 0, 2),
        ((0, 0), (0, n_rows - n_nodes), (0, 0))).reshape(
            _NUM_CORES, n_pairs, feat)
    acc = _sc_aggregate(x2p, sdw, n_pairs, feat)
    acc_halves = acc.reshape(_NUM_CORES, n_rows, feat_half)
    W2 = W.reshape(_NUM_CORES, feat_half, W.shape[1])
    out = _tc_matmul_relu(acc_halves, W2)
    return out[:n_nodes]


# speedup vs baseline: 1.1001x; 1.0085x over previous
import jax
import jax.numpy as jnp
from jax import lax
from jax.experimental import pallas as pl
from jax.experimental.pallas import tpu as pltpu
from jax.experimental.pallas import tpu_sc as plsc

_NUM_CORES = 2
_NUM_SUBCORES = 16
_LANES = 16
_CHUNK = 128
_SLOTS = 4


def _sc_aggregate(x2p, sdw, n_pairs, feat):
    n_chunks = sdw.shape[1]
    pairs_per_tile = n_pairs // _NUM_SUBCORES
    half = feat // 2
    jblocks = half // _LANES

    mesh = plsc.VectorSubcoreMesh(
        core_axis_name="c", subcore_axis_name="s",
        num_cores=_NUM_CORES, num_subcores=_NUM_SUBCORES)

    def body(x_hbm, sdw_hbm, out_hbm,
             xs, acc, sl0, sl1, sl2, sl3, rows0, rows1, dst0, dst1,
             gsem0, gsem1, ssem0, ssem1, ssem2, ssem3, scsem0, scsem1):
        cid = lax.axis_index("c")
        sid = lax.axis_index("s")
        slots = (sl0, sl1, sl2, sl3)
        slot_sems = (ssem0, ssem1, ssem2, ssem3)
        rows = (rows0, rows1)
        gsems = (gsem0, gsem1)
        dsts = (dst0, dst1)
        scsems = (scsem0, scsem1)

        def wait_scatter(b):
            pltpu.make_async_copy(rows[b], acc.at[dsts[b]], scsems[b]).wait()

        def load_sdw(ci, t):
            pltpu.async_copy(sdw_hbm.at[sid, ci], slots[t], slot_sems[t])

        def wait_sdw(ci, t):
            pltpu.make_async_copy(
                sdw_hbm.at[sid, ci], slots[t], slot_sems[t]).wait()

        def start_gather(t, b):
            pltpu.async_copy(xs.at[slots[t].at[0]], rows[b], gsems[b])

        def wait_gather(t, b):
            pltpu.make_async_copy(
                xs.at[slots[t].at[0]], rows[b], gsems[b]).wait()

        for t in range(_SLOTS):
            load_sdw(t, t)

        row0 = sid * pairs_per_tile
        pltpu.sync_copy(x_hbm.at[cid, pl.ds(row0, pairs_per_tile)],
                        xs.at[pl.ds(row0, pairs_per_tile)])

        zvec = jnp.zeros((_LANES,), jnp.float32)

        def zfill(i, _):
            for j in range(feat // _LANES):
                rows0[i, pl.ds(j * _LANES, _LANES)] = zvec
            return 0

        lax.fori_loop(0, _CHUNK, zfill, 0)
        full, rem = divmod(pairs_per_tile, _CHUNK)
        for k in range(full):
            pltpu.sync_copy(rows0, acc.at[pl.ds(row0 + k * _CHUNK, _CHUNK)])
        if rem:
            pltpu.sync_copy(rows0.at[pl.ds(0, rem)],
                            acc.at[pl.ds(row0 + full * _CHUNK, rem)])
        plsc.subcore_barrier()
        wait_sdw(0, 0)
        start_gather(0, 0)

        def scale(t, b):
            buf = rows[b]
            wrow = slots[t]

            def sbody(g, _):
                w16 = lax.bitcast_convert_type(
                    wrow[2, pl.ds(g * _LANES, _LANES)], jnp.float32)
                sp16 = wrow[3, pl.ds(g * _LANES, _LANES)]
                dp16 = wrow[4, pl.ds(g * _LANES, _LANES)]
                for k in range(_LANES):
                    e = g * _LANES + k
                    we = w16[k]
                    sodd = sp16[k] != 0
                    dodd = dp16[k] != 0
                    msg = []
                    for j in range(jblocks):
                        a = buf[e, pl.ds(j * _LANES, _LANES)]
                        bb = buf[e, pl.ds(half + j * _LANES, _LANES)]
                        msg.append(jnp.where(sodd, bb, a) * we)
                    for j in range(jblocks):
                        buf[e, pl.ds(j * _LANES, _LANES)] = (
                            jnp.where(dodd, zvec, msg[j]))
                        buf[e, pl.ds(half + j * _LANES, _LANES)] = (
                            jnp.where(dodd, msg[j], zvec))
                return 0

            lax.fori_loop(0, _CHUNK // _LANES, sbody, 0)

        def quad(q, _):
            c0 = q * _SLOTS
            for t in range(_SLOTS):
                ct = c0 + t
                b = t % 2
                if t < _SLOTS - 1:
                    wait_sdw(ct + 1, t + 1)

                    @pl.when(ct >= 1)
                    def _():
                        wait_scatter(1 - b)

                    start_gather(t + 1, 1 - b)
                else:
                    @pl.when(ct + 1 < n_chunks)
                    def _():
                        wait_sdw(ct + 1, 0)
                        wait_scatter(1 - b)
                        start_gather(0, 1 - b)

                wait_gather(t, b)
                scale(t, b)
                for g in range(_CHUNK // _LANES):
                    dsts[b][pl.ds(g * _LANES, _LANES)] = (
                        slots[t][1, pl.ds(g * _LANES, _LANES)])
                pltpu.async_copy(rows[b], acc.at[dsts[b]], scsems[b],
                                 add=True)

                @pl.when(ct + _SLOTS < n_chunks)
                def _():
                    load_sdw(ct + _SLOTS, t)
            return 0

        lax.fori_loop(0, n_chunks // _SLOTS, quad, 0)
        wait_scatter(0)
        wait_scatter(1)
        plsc.subcore_barrier()

        pltpu.sync_copy(acc.at[pl.ds(row0, pairs_per_tile)],
                        out_hbm.at[cid, pl.ds(row0, pairs_per_tile)])

    fn = pl.kernel(
        body,
        out_type=jax.ShapeDtypeStruct((_NUM_CORES, n_pairs, feat),
                                      jnp.float32),
        mesh=mesh,
        scratch_types=[
            pltpu.VMEM_SHARED((n_pairs, feat), jnp.float32),
            pltpu.VMEM_SHARED((n_pairs, feat), jnp.float32),
            pltpu.VMEM((5, _CHUNK), jnp.int32),
            pltpu.VMEM((5, _CHUNK), jnp.int32),
            pltpu.VMEM((5, _CHUNK), jnp.int32),
            pltpu.VMEM((5, _CHUNK), jnp.int32),
            pltpu.VMEM((_CHUNK, feat), jnp.float32),
            pltpu.VMEM((_CHUNK, feat), jnp.float32),
            pltpu.VMEM((_CHUNK,), jnp.int32),
            pltpu.VMEM((_CHUNK,), jnp.int32),
            pltpu.SemaphoreType.DMA,
            pltpu.SemaphoreType.DMA,
            pltpu.SemaphoreType.DMA,
            pltpu.SemaphoreType.DMA,
            pltpu.SemaphoreType.DMA,
            pltpu.SemaphoreType.DMA,
            pltpu.SemaphoreType.DMA,
            pltpu.SemaphoreType.DMA,
        ],
    )
    return fn(x2p, sdw)


def _tc_matmul_relu(acc, W2):
    n_rows, feat_half = acc.shape[1], acc.shape[2]
    out_f = W2.shape[2]
    block = 1024
    grid = n_rows // block

    def body(a_ref, w_ref, o_ref):
        o_ref[...] = jnp.maximum(
            jnp.dot(a_ref[0], w_ref[0], preferred_element_type=jnp.float32)
            + jnp.dot(a_ref[1], w_ref[1], preferred_element_type=jnp.float32),
            0.0)

    return pl.pallas_call(
        body,
        grid=(grid,),
        in_specs=[
            pl.BlockSpec((_NUM_CORES, block, feat_half), lambda i: (0, i, 0)),
            pl.BlockSpec((_NUM_CORES, feat_half, out_f), lambda i: (0, 0, 0)),
        ],
        out_specs=pl.BlockSpec((block, out_f), lambda i: (i, 0)),
        out_shape=jax.ShapeDtypeStruct((n_rows, out_f), jnp.float32),
    )(acc, W2)


def kernel(x, edge_index, edge_weight, W):
    n_nodes, feat = x.shape
    n_edges = edge_index.shape[1]
    feat_half = feat // _NUM_CORES
    grain = _NUM_SUBCORES * _CHUNK * _SLOTS
    e_pad = -(-n_edges // grain) * grain
    pad = e_pad - n_edges
    src = jnp.concatenate([edge_index[0], jnp.zeros((pad,), jnp.int32)])
    dst = jnp.concatenate([edge_index[1], jnp.zeros((pad,), jnp.int32)])
    w = jnp.concatenate([edge_weight, jnp.zeros((pad,), jnp.float32)])
    n_chunks = e_pad // (_NUM_SUBCORES * _CHUNK)
    shape3 = (_NUM_SUBCORES, n_chunks, _CHUNK)
    sdw = jnp.stack([
        (src >> 1).reshape(shape3),
        (dst >> 1).reshape(shape3),
        lax.bitcast_convert_type(w, jnp.int32).reshape(shape3),
        (src & 1).reshape(shape3),
        (dst & 1).reshape(shape3),
    ], axis=2)
    rgrain = _NUM_SUBCORES * _CHUNK
    n_rows = -(-n_nodes // rgrain) * rgrain
    n_pairs = n_rows // 2
    x2p = jnp.pad(
        x.reshape(n_nodes, _NUM_CORES, feat_half).transpose(1, 0, 2),
        ((0, 0), (0, n_rows - n_nodes), (0, 0))).reshape(
            _NUM_CORES, n_pairs, feat)
    acc = _sc_aggregate(x2p, sdw, n_pairs, feat)
    acc_halves = acc.reshape(_NUM_CORES, n_rows, feat_half)
    W2 = W.reshape(_NUM_CORES, feat_half, W.shape[1])
    out = _tc_matmul_relu(acc_halves, W2)
    return out[:n_nodes]
